# baseline (speedup 1.0000x reference)
import jax
import jax.numpy as jnp
from jax import lax
from jax.experimental import pallas as pl
from jax.experimental.pallas import tpu as pltpu
from jax.experimental.pallas import tpu_sc as plsc

_NUM_CORES = 2
_NUM_SUBCORES = 16
_NUM_WORKERS = _NUM_CORES * _NUM_SUBCORES

_BI = 512


def _fire(table_hbm, idxt_hbm, u, idx_v, rows_v, gsem):
    pltpu.sync_copy(idxt_hbm.at[u], idx_v)
    for q in range(4):
        pltpu.async_copy(table_hbm.at[idx_v.at[q]], rows_v.at[q], gsem)


def _drain(table_hbm, idx_v, rows_v, gsem):
    for q in range(4):
        pltpu.make_async_copy(
            table_hbm.at[idx_v.at[q]], rows_v.at[q], gsem).wait()


def _transpose(rows_v, t_v, sid, tsem):
    for c in range(32):
        pltpu.async_copy(rows_v.at[:, :, c], t_v.at[sid, c], tsem)
    for c in range(32):
        pltpu.make_async_copy(rows_v.at[:, :, c], t_v.at[sid, c], tsem).wait()


def _write(outp_hbm, u, t_v, sid, wsem):
    j = u // 32
    ib = u % 32
    for c in range(32):
        kg, ks = c // 8, c % 8
        pltpu.async_copy(
            t_v.at[sid, c], outp_hbm.at[j, kg, pl.ds(ib * 4, 4), ks], wsem)


def _wait_write(outp_hbm, t_v, sid, wsem):
    for c in range(32):
        kg, ks = c // 8, c % 8
        pltpu.make_async_copy(
            t_v.at[sid, c], outp_hbm.at[0, kg, pl.ds(0, 4), ks], wsem).wait()


def _gather_t_kernel(table_hbm, idxt_hbm, outp_hbm,
                     idx0, idx1, rows0, rows1, t0, t1,
                     g0, g1, w0, w1, tsem):
    sid = lax.axis_index("s")
    wid = sid * _NUM_CORES + lax.axis_index("c")
    units_per_worker = 6400 // _NUM_WORKERS
    n_pairs = units_per_worker // 2
    u_base = wid * units_per_worker

    _fire(table_hbm, idxt_hbm, u_base, idx0, rows0, g0)

    def body(p, carry):
        u0 = u_base + 2 * p
        u1 = u0 + 1
        u2 = u0 + 2

        _fire(table_hbm, idxt_hbm, u1, idx1, rows1, g1)

        _drain(table_hbm, idx0, rows0, g0)

        @pl.when(p > 0)
        def _():
            _wait_write(outp_hbm, t0, sid, w0)

        _transpose(rows0, t0, sid, tsem)
        _write(outp_hbm, u0, t0, sid, w0)

        @pl.when(p < n_pairs - 1)
        def _():
            _fire(table_hbm, idxt_hbm, u2, idx0, rows0, g0)

        _drain(table_hbm, idx1, rows1, g1)

        @pl.when(p > 0)
        def _():
            _wait_write(outp_hbm, t1, sid, w1)

        _transpose(rows1, t1, sid, tsem)
        _write(outp_hbm, u1, t1, sid, w1)
        return carry

    lax.fori_loop(0, n_pairs, body, 0, unroll=False)

    _wait_write(outp_hbm, t0, sid, w0)
    _wait_write(outp_hbm, t1, sid, w1)


def kernel(a, idx):
    d = a.shape[1]
    idxt = idx.T.reshape(6400, 4, 128).astype(jnp.int32)

    mesh = plsc.VectorSubcoreMesh(
        core_axis_name="c", subcore_axis_name="s",
        num_cores=_NUM_CORES, num_subcores=_NUM_SUBCORES,
    )
    k = pl.kernel(
        _gather_t_kernel,
        out_type=jax.ShapeDtypeStruct((200, 4, 128, 8, 128), jnp.float32),
        mesh=mesh,
        scratch_types=[
            pltpu.VMEM((4, 128), jnp.int32),
            pltpu.VMEM((4, 128), jnp.int32),
            pltpu.VMEM((4, 128, d), jnp.float32),
            pltpu.VMEM((4, 128, d), jnp.float32),
            pltpu.VMEM_SHARED((_NUM_SUBCORES, d, 4, 128), jnp.float32),
            pltpu.VMEM_SHARED((_NUM_SUBCORES, d, 4, 128), jnp.float32),
            pltpu.SemaphoreType.DMA,
            pltpu.SemaphoreType.DMA,
            pltpu.SemaphoreType.DMA,
            pltpu.SemaphoreType.DMA,
            pltpu.SemaphoreType.DMA,
        ],
        compiler_params=pltpu.CompilerParams(use_tc_tiling_on_sc=False),
    )
    outp = k(a, idxt)
    out = outp.transpose(2, 4, 0, 1, 3)
    return out.reshape(idx.shape + (d,))


# speedup vs baseline: 4.9005x; 4.9005x over previous
import functools

import jax
import jax.numpy as jnp
from jax import lax
from jax.experimental import pallas as pl
from jax.experimental.pallas import tpu as pltpu
from jax.experimental.pallas import tpu_sc as plsc

_NUM_CORES = 2
_NUM_SUBCORES = 16
_NUM_WORKERS = _NUM_CORES * _NUM_SUBCORES

_CHUNK = 1024
_NSTREAM = 4
_SUB = _CHUNK // _NSTREAM


def _fire_gather(table_hbm, idx_v, rows_v, sem):
    for s in range(_NSTREAM):
        pltpu.async_copy(
            table_hbm.at[idx_v.at[pl.ds(s * _SUB, _SUB)]],
            rows_v.at[pl.ds(s * _SUB, _SUB)], sem)


def _drain_gather(table_hbm, idx_v, rows_v, sem):
    for s in range(_NSTREAM):
        pltpu.make_async_copy(
            table_hbm.at[idx_v.at[pl.ds(s * _SUB, _SUB)]],
            rows_v.at[pl.ds(s * _SUB, _SUB)], sem).wait()


def _gather_kernel(n_rows, d, table_hbm, idx_hbm, out_hbm,
                   idx0, idx1, rows0, rows1, g0, g1, w0, w1):
    rows_per_worker = n_rows // _NUM_WORKERS
    n_chunks = rows_per_worker // _CHUNK
    n_pairs = n_chunks // 2
    wid = lax.axis_index("s") * _NUM_CORES + lax.axis_index("c")
    worker_base = wid * rows_per_worker

    pltpu.sync_copy(idx_hbm.at[pl.ds(worker_base, _CHUNK)], idx0)
    _fire_gather(table_hbm, idx0, rows0, g0)

    def body(i, carry):
        base0 = worker_base + (2 * i) * _CHUNK
        base1 = base0 + _CHUNK
        base2 = base1 + _CHUNK

        pltpu.sync_copy(idx_hbm.at[pl.ds(base1, _CHUNK)], idx1)

        @pl.when(i > 0)
        def _():
            pltpu.make_async_copy(
                rows1, out_hbm.at[pl.ds(base1 - 2 * _CHUNK, _CHUNK)], w1).wait()

        _fire_gather(table_hbm, idx1, rows1, g1)

        _drain_gather(table_hbm, idx0, rows0, g0)
        pltpu.async_copy(rows0, out_hbm.at[pl.ds(base0, _CHUNK)], w0)

        @pl.when(i < n_pairs - 1)
        def _():
            pltpu.sync_copy(idx_hbm.at[pl.ds(base2, _CHUNK)], idx0)

        pltpu.make_async_copy(rows0, out_hbm.at[pl.ds(base0, _CHUNK)], w0).wait()

        @pl.when(i < n_pairs - 1)
        def _():
            _fire_gather(table_hbm, idx0, rows0, g0)

        _drain_gather(table_hbm, idx1, rows1, g1)
        pltpu.async_copy(rows1, out_hbm.at[pl.ds(base1, _CHUNK)], w1)
        return carry

    lax.fori_loop(0, n_pairs, body, 0, unroll=False)

    last_base = worker_base + (n_chunks - 1) * _CHUNK
    pltpu.make_async_copy(rows1, out_hbm.at[pl.ds(last_base, _CHUNK)], w1).wait()


def _sc_gather(a, idx_flat):
    n_rows = idx_flat.shape[0]
    d = a.shape[1]
    mesh = plsc.VectorSubcoreMesh(
        core_axis_name="c", subcore_axis_name="s",
        num_cores=_NUM_CORES, num_subcores=_NUM_SUBCORES,
    )
    k = pl.kernel(
        functools.partial(_gather_kernel, n_rows, d),
        out_type=jax.ShapeDtypeStruct((n_rows, d), jnp.float32),
        mesh=mesh,
        scratch_types=[
            pltpu.VMEM((_CHUNK,), jnp.int32),
            pltpu.VMEM((_CHUNK,), jnp.int32),
            pltpu.VMEM((_CHUNK, d), jnp.float32),
            pltpu.VMEM((_CHUNK, d), jnp.float32),
            pltpu.SemaphoreType.DMA,
            pltpu.SemaphoreType.DMA,
            pltpu.SemaphoreType.DMA,
            pltpu.SemaphoreType.DMA,
        ],
        compiler_params=pltpu.CompilerParams(use_tc_tiling_on_sc=False),
    )
    return k(a, idx_flat)


def _retile_kernel(in_ref, out_ref):
    for t in range(16):
        v = in_ref[pl.ds(t * 32, 32), :]
        w = v.reshape(128, 32).T
        out_ref[0, :, t, :, :] = w.reshape(4, 8, 128)


def _tc_retile(y):
    return pl.pallas_call(
        _retile_kernel,
        grid=(200, 8),
        in_specs=[pl.BlockSpec((512, 128), lambda j, g: (j * 8 + g, 0))],
        out_specs=pl.BlockSpec(
            (1, 4, 16, 8, 128), lambda j, g: (j, 0, g, 0, 0)),
        out_shape=jax.ShapeDtypeStruct((200, 4, 128, 8, 128), jnp.float32),
    )(y)


def kernel(a, idx):
    d = a.shape[1]
    idx_flat = idx.T.reshape(-1).astype(jnp.int32)
    outl = _sc_gather(a, idx_flat)
    y = outl.reshape(outl.shape[0] // 4, 4 * d)
    outp = _tc_retile(y)
    out = outp.transpose(2, 4, 0, 1, 3)
    return out.reshape(idx.shape + (d,))
